# baseline (speedup 1.0000x reference)
import jax
import jax.numpy as jnp
from jax import lax
from jax.experimental import pallas as pl
from jax.experimental.pallas import tpu as pltpu
from jax.experimental.pallas import tpu_sc as plsc

BATCH = 2048
IN_FEATURES = 1024
WINDOW = 128
NK = 56
NUM_WINDOWS = 14
LANES = 16
CHUNKS = WINDOW // LANES

NUM_CORES = 2
NUM_SUBCORES = 16
NUM_WORKERS = NUM_CORES * NUM_SUBCORES

SC_ROWS = 1024
TC_ROWS = BATCH - SC_ROWS
SC_WROWS = SC_ROWS // 2
WROWS_PER_WORKER = SC_WROWS // NUM_WORKERS
WROW_GROUPS = WROWS_PER_WORKER // LANES
ROWS_PER_WORKER = SC_ROWS // NUM_WORKERS

CAST_BLOCK = 128
MM_BLOCK = 256

_ILV = plsc.PackFormat.INTERLEAVED
_STARTS = tuple(64 * j for j in range(13)) + (896,)


def _rne_bf16_bits(u):
    lsb = lax.shift_right_logical(u, jnp.uint32(16)) & jnp.uint32(1)
    return (u + jnp.uint32(0x7FFF) + lsb) & jnp.uint32(0xFFFF0000)


def _pack_body(xp_ref, w_ref, xo_ref, wo_ref):
    xp = lax.bitcast_convert_type(xp_ref[...], jnp.uint32)
    lo = _rne_bf16_bits(xp[:, :IN_FEATURES])
    hi = _rne_bf16_bits(xp[:, IN_FEATURES:])
    word = lax.shift_right_logical(lo, jnp.uint32(16)) | hi
    xo_ref[...] = lax.bitcast_convert_type(word, jnp.int32)
    wu = _rne_bf16_bits(lax.bitcast_convert_type(w_ref[...], jnp.uint32))
    wd = lax.shift_right_logical(wu, jnp.uint32(16)) | wu
    wo_ref[...] = lax.bitcast_convert_type(wd, jnp.int32)


def _pack(xp, weight):
    return pl.pallas_call(
        _pack_body,
        grid=(SC_WROWS // CAST_BLOCK,),
        in_specs=[
            pl.BlockSpec((CAST_BLOCK, 2 * IN_FEATURES), lambda i: (i, 0)),
            pl.BlockSpec((NK, WINDOW), lambda i: (0, 0)),
        ],
        out_specs=[
            pl.BlockSpec((CAST_BLOCK, IN_FEATURES), lambda i: (i, 0)),
            pl.BlockSpec((NK, WINDOW), lambda i: (0, 0)),
        ],
        out_shape=[
            jax.ShapeDtypeStruct((SC_WROWS, IN_FEATURES), jnp.int32),
            jax.ShapeDtypeStruct((NK, WINDOW), jnp.int32),
        ],
    )(xp, weight)


def _mm_body(x_ref, w_ref, o_ref):
    xb = x_ref[...]
    w = w_ref[...]
    outs = []
    for j in range(NUM_WINDOWS):
        s = _STARTS[j]
        outs.append(
            lax.dot_general(
                xb[:, s : s + WINDOW],
                w[4 * j : 4 * j + 4, :],
                (((1,), (1,)), ((), ())),
                preferred_element_type=jnp.float32,
            )
        )
    o_ref[...] = jnp.concatenate(outs, axis=1)


def _mm(x, weight):
    off = SC_ROWS // MM_BLOCK
    return pl.pallas_call(
        _mm_body,
        grid=(TC_ROWS // MM_BLOCK,),
        in_specs=[
            pl.BlockSpec((MM_BLOCK, IN_FEATURES), lambda i: (i + off, 0)),
            pl.BlockSpec((NK, WINDOW), lambda i: (0, 0)),
        ],
        out_specs=pl.BlockSpec((MM_BLOCK, NK), lambda i: (i, 0)),
        out_shape=jax.ShapeDtypeStruct((TC_ROWS, NK), jnp.float32),
    )(x, weight)


def _fbl_body(xw_hbm, wd_hbm, out_hbm, xw_v, wd_v, out_v, st0, st1, st2, st3):
    wid = lax.axis_index("s") * NUM_CORES + lax.axis_index("c")
    wbase = wid * WROWS_PER_WORKER

    pltpu.sync_copy(xw_hbm.at[pl.ds(wbase, WROWS_PER_WORKER)], xw_v)
    pltpu.sync_copy(wd_hbm, wd_v)

    lanes = lax.iota(jnp.int32, LANES)
    gidx = lanes * LANES
    sidx = lanes * (2 * NK)
    stage = [st0, st1, st2, st3]

    def load_row(r, start):
        return [
            plsc.bitcast(
                xw_v[r, pl.ds(start + LANES * k, LANES)], jnp.bfloat16
            )
            for k in range(CHUNKS)
        ]

    def step(j, _):
        start = lax.select(
            j == NUM_WINDOWS - 1,
            jnp.int32(IN_FEATURES - WINDOW),
            jnp.int32(64) * j,
        )
        wv = [
            [
                plsc.bitcast(
                    wd_v[4 * j + c, pl.ds(LANES * k, LANES)], jnp.bfloat16
                )
                for k in range(CHUNKS)
            ]
            for c in range(4)
        ]
        xcur = load_row(0, start)
        for wl in range(LANES):
            xnxt = load_row(wl + 1, start) if wl < LANES - 1 else None
            for c in range(4):
                w8 = wv[c]
                acc = (
                    (xcur[0] * w8[0] + xcur[1] * w8[1])
                    + (xcur[2] * w8[2] + xcur[3] * w8[3])
                ) + (
                    (xcur[4] * w8[4] + xcur[5] * w8[5])
                    + (xcur[6] * w8[6] + xcur[7] * w8[7])
                )
                stage[c][pl.ds(wl * LANES, LANES)] = plsc.bitcast(
                    acc, jnp.int32
                )
            xcur = xnxt
        for c in range(4):
            cols = [
                plsc.bitcast(
                    plsc.load_gather(stage[c], [gidx + l]), jnp.bfloat16
                )
                for l in range(LANES)
            ]
            while len(cols) > 1:
                cols = [
                    cols[2 * i] + cols[2 * i + 1]
                    for i in range(len(cols) // 2)
                ]
            even, odd = plsc.unpack(cols[0], format=_ILV)
            off = 4 * j + c
            plsc.store_scatter(out_v, [sidx + off], even)
            plsc.store_scatter(out_v, [sidx + (off + NK)], odd)
        return _

    lax.fori_loop(0, NUM_WINDOWS, step, None)

    pltpu.sync_copy(
        out_v, out_hbm.at[pl.ds(wbase * 2 * NK, ROWS_PER_WORKER * NK)]
    )


@jax.jit
def _fbl(x, weight):
    xw, wd = _pack(x.reshape(BATCH // 2, 2 * IN_FEATURES), weight)
    tc_out = _mm(x, weight)
    mesh = plsc.VectorSubcoreMesh(
        core_axis_name="c",
        subcore_axis_name="s",
        num_cores=NUM_CORES,
        num_subcores=NUM_SUBCORES,
    )
    run = pl.kernel(
        _fbl_body,
        out_type=jax.ShapeDtypeStruct((SC_ROWS * NK,), jnp.float32),
        mesh=mesh,
        scratch_types=[
            pltpu.VMEM((WROWS_PER_WORKER, IN_FEATURES), jnp.int32),
            pltpu.VMEM((NK, WINDOW), jnp.int32),
            pltpu.VMEM((ROWS_PER_WORKER * NK,), jnp.float32),
            pltpu.VMEM((LANES * LANES,), jnp.int32),
            pltpu.VMEM((LANES * LANES,), jnp.int32),
            pltpu.VMEM((LANES * LANES,), jnp.int32),
            pltpu.VMEM((LANES * LANES,), jnp.int32),
        ],
        compiler_params=pltpu.CompilerParams(needs_layout_passes=False),
    )
    sc_out = run(xw, wd).reshape(SC_ROWS, NK)
    return jnp.concatenate([sc_out, tc_out], axis=0)


def kernel(x, weight, fbank):
    del fbank
    return _fbl(x, weight)


# speedup vs baseline: 1.3034x; 1.3034x over previous
import jax
import jax.numpy as jnp
from jax import lax
from jax.experimental import pallas as pl
from jax.experimental.pallas import tpu as pltpu
from jax.experimental.pallas import tpu_sc as plsc

BATCH = 2048
IN_FEATURES = 1024
WINDOW = 128
NK = 56
NUM_WINDOWS = 14
LANES = 16
CHUNKS = WINDOW // LANES

NUM_CORES = 2
NUM_SUBCORES = 16
NUM_WORKERS = NUM_CORES * NUM_SUBCORES

SC_ROWS = 1024
TC_ROWS = BATCH - SC_ROWS
SC_WROWS = SC_ROWS // 2
WROWS_PER_WORKER = SC_WROWS // NUM_WORKERS
WROW_GROUPS = WROWS_PER_WORKER // LANES
ROWS_PER_WORKER = SC_ROWS // NUM_WORKERS

CAST_BLOCK = 128
MM_BLOCK = 256

_ILV = plsc.PackFormat.INTERLEAVED
_STARTS = tuple(64 * j for j in range(13)) + (896,)


def _rne_bf16_bits(u):
    lsb = lax.shift_right_logical(u, jnp.uint32(16)) & jnp.uint32(1)
    return (u + jnp.uint32(0x7FFF) + lsb) & jnp.uint32(0xFFFF0000)


def _pack_body(x_ref, w_ref, xo_ref, wo_ref):
    xu = lax.bitcast_convert_type(x_ref[...], jnp.uint32)
    lo = _rne_bf16_bits(xu[:CAST_BLOCK, :])
    hi = _rne_bf16_bits(xu[CAST_BLOCK:, :])
    word = lax.shift_right_logical(lo, jnp.uint32(16)) | hi
    xo_ref[...] = lax.bitcast_convert_type(word, jnp.int32)
    wu = _rne_bf16_bits(lax.bitcast_convert_type(w_ref[...], jnp.uint32))
    wd = lax.shift_right_logical(wu, jnp.uint32(16)) | wu
    wo_ref[...] = lax.bitcast_convert_type(wd, jnp.int32)


def _pack(x, weight):
    return pl.pallas_call(
        _pack_body,
        grid=(SC_ROWS // (2 * CAST_BLOCK),),
        in_specs=[
            pl.BlockSpec((2 * CAST_BLOCK, IN_FEATURES), lambda i: (i, 0)),
            pl.BlockSpec((NK, WINDOW), lambda i: (0, 0)),
        ],
        out_specs=[
            pl.BlockSpec((CAST_BLOCK, IN_FEATURES), lambda i: (i, 0)),
            pl.BlockSpec((NK, WINDOW), lambda i: (0, 0)),
        ],
        out_shape=[
            jax.ShapeDtypeStruct((SC_WROWS, IN_FEATURES), jnp.int32),
            jax.ShapeDtypeStruct((NK, WINDOW), jnp.int32),
        ],
    )(x, weight)


def _mm_body(x_ref, w_ref, o_ref):
    xb = x_ref[...]
    w = w_ref[...]
    outs = []
    for j in range(NUM_WINDOWS):
        s = _STARTS[j]
        outs.append(
            lax.dot_general(
                xb[:, s : s + WINDOW],
                w[4 * j : 4 * j + 4, :],
                (((1,), (1,)), ((), ())),
                preferred_element_type=jnp.float32,
            )
        )
    o_ref[...] = jnp.concatenate(outs, axis=1)


def _mm(x, weight):
    off = SC_ROWS // MM_BLOCK
    return pl.pallas_call(
        _mm_body,
        grid=(TC_ROWS // MM_BLOCK,),
        in_specs=[
            pl.BlockSpec((MM_BLOCK, IN_FEATURES), lambda i: (i + off, 0)),
            pl.BlockSpec((NK, WINDOW), lambda i: (0, 0)),
        ],
        out_specs=pl.BlockSpec((MM_BLOCK, NK), lambda i: (i, 0)),
        out_shape=jax.ShapeDtypeStruct((TC_ROWS, NK), jnp.float32),
    )(x, weight)


def _fbl_body(xw_hbm, wd_hbm, out_hbm, xw_v, wd_v, out_v, st0, st1, st2, st3):
    wid = lax.axis_index("s") * NUM_CORES + lax.axis_index("c")
    wbase = wid * WROWS_PER_WORKER

    pltpu.sync_copy(xw_hbm.at[pl.ds(wbase, WROWS_PER_WORKER)], xw_v)
    pltpu.sync_copy(wd_hbm, wd_v)

    lanes = lax.iota(jnp.int32, LANES)
    gidx = lanes * LANES
    sidx = lanes * NK
    stage = [st0, st1, st2, st3]

    def load_row(r, start):
        return [
            plsc.bitcast(
                xw_v[r, pl.ds(start + LANES * k, LANES)], jnp.bfloat16
            )
            for k in range(CHUNKS)
        ]

    def step(j, _):
        start = lax.select(
            j == NUM_WINDOWS - 1,
            jnp.int32(IN_FEATURES - WINDOW),
            jnp.int32(64) * j,
        )
        wv = [
            [
                plsc.bitcast(
                    wd_v[4 * j + c, pl.ds(LANES * k, LANES)], jnp.bfloat16
                )
                for k in range(CHUNKS)
            ]
            for c in range(4)
        ]
        xcur = load_row(0, start)
        for wl in range(LANES):
            xnxt = load_row(wl + 1, start) if wl < LANES - 1 else None
            for c in range(4):
                w8 = wv[c]
                acc = (
                    (xcur[0] * w8[0] + xcur[1] * w8[1])
                    + (xcur[2] * w8[2] + xcur[3] * w8[3])
                ) + (
                    (xcur[4] * w8[4] + xcur[5] * w8[5])
                    + (xcur[6] * w8[6] + xcur[7] * w8[7])
                )
                stage[c][pl.ds(wl * LANES, LANES)] = plsc.bitcast(
                    acc, jnp.int32
                )
            xcur = xnxt
        for c in range(4):
            cols = [
                plsc.bitcast(
                    plsc.load_gather(stage[c], [gidx + l]), jnp.bfloat16
                )
                for l in range(LANES)
            ]
            while len(cols) > 1:
                cols = [
                    cols[2 * i] + cols[2 * i + 1]
                    for i in range(len(cols) // 2)
                ]
            even, odd = plsc.unpack(cols[0], format=_ILV)
            off = 4 * j + c
            plsc.store_scatter(out_v, [sidx + off], even)
            plsc.store_scatter(out_v, [sidx + (LANES * NK + off)], odd)
        return _

    lax.fori_loop(0, NUM_WINDOWS, step, None)

    blk = wid // 8
    rowa = blk * 256 + (wid % 8) * LANES
    pltpu.sync_copy(
        out_v.at[pl.ds(0, LANES * NK)],
        out_hbm.at[pl.ds(rowa * NK, LANES * NK)],
    )
    pltpu.sync_copy(
        out_v.at[pl.ds(LANES * NK, LANES * NK)],
        out_hbm.at[pl.ds((rowa + 128) * NK, LANES * NK)],
    )


@jax.jit
def _fbl(x, weight):
    xw, wd = _pack(x, weight)
    tc_out = _mm(x, weight)
    mesh = plsc.VectorSubcoreMesh(
        core_axis_name="c",
        subcore_axis_name="s",
        num_cores=NUM_CORES,
        num_subcores=NUM_SUBCORES,
    )
    run = pl.kernel(
        _fbl_body,
        out_type=jax.ShapeDtypeStruct((SC_ROWS * NK,), jnp.float32),
        mesh=mesh,
        scratch_types=[
            pltpu.VMEM((WROWS_PER_WORKER, IN_FEATURES), jnp.int32),
            pltpu.VMEM((NK, WINDOW), jnp.int32),
            pltpu.VMEM((ROWS_PER_WORKER * NK,), jnp.float32),
            pltpu.VMEM((LANES * LANES,), jnp.int32),
            pltpu.VMEM((LANES * LANES,), jnp.int32),
            pltpu.VMEM((LANES * LANES,), jnp.int32),
            pltpu.VMEM((LANES * LANES,), jnp.int32),
        ],
        compiler_params=pltpu.CompilerParams(needs_layout_passes=False),
    )
    sc_out = run(xw, wd).reshape(SC_ROWS, NK)
    return jnp.concatenate([sc_out, tc_out], axis=0)


def kernel(x, weight, fbank):
    del fbank
    return _fbl(x, weight)
